# Initial kernel scaffold; baseline (speedup 1.0000x reference)
import jax
import jax.numpy as jnp
from jax.experimental import pallas as pl


def kernel(node_features, gathered_nodes, edge_list):
    raise NotImplementedError("write your pallas kernel here")


# speedup vs baseline: 5.0828x; 5.0828x over previous
import functools

import jax
import jax.numpy as jnp
from jax import lax
from jax.experimental import pallas as pl
from jax.experimental.pallas import tpu as pltpu
from jax.experimental.pallas import tpu_sc as plsc

_N_NODES = 10000
_N_EDGES = 320000
_D = 128

_NC = 2
_NS = 16
_NW = _NC * _NS

_EPT = _N_EDGES // _NW
_CHUNK = 128
_FULL_CHUNKS = _EPT // _CHUNK
_REM = _EPT - _FULL_CHUNKS * _CHUNK

_N_PAD = 10240
_ROWS_PER_TILE = _N_PAD // _NS
_ZROWS = 128


def _sc_scatter_add(gathered_hbm, dst_hbm, out_hbm, acc, feat_v, idx_v,
                    feat16_v, idx16_v, zbuf_v):
  c = lax.axis_index("c")
  s = lax.axis_index("s")
  w = c * _NS + s

  z = jnp.zeros((16,), jnp.float32)

  def zero_row(r, carry):
    for l in range(_D // 16):
      zbuf_v[r, pl.ds(16 * l, 16)] = z
    return carry

  lax.fori_loop(0, _ZROWS, zero_row, 0)
  row0 = s * _ROWS_PER_TILE
  for k in range(_ROWS_PER_TILE // _ZROWS):
    pltpu.sync_copy(zbuf_v, acc.at[pl.ds(row0 + k * _ZROWS, _ZROWS)])
  plsc.subcore_barrier()

  base = w * _EPT

  def chunk_body(k, carry):
    e0 = pl.multiple_of(base + k * _CHUNK, 16)
    pltpu.sync_copy(dst_hbm.at[pl.ds(e0, _CHUNK)], idx_v)
    pltpu.sync_copy(gathered_hbm.at[pl.ds(e0, _CHUNK), 0], feat_v)
    pltpu.sync_copy(feat_v, acc.at[idx_v], add=True)
    return carry

  lax.fori_loop(0, _FULL_CHUNKS, chunk_body, 0)

  e0 = pl.multiple_of(base + _FULL_CHUNKS * _CHUNK, 8)
  pltpu.sync_copy(dst_hbm.at[pl.ds(e0, _REM)], idx16_v)
  pltpu.sync_copy(gathered_hbm.at[pl.ds(e0, _REM), 0], feat16_v)
  pltpu.sync_copy(feat16_v, acc.at[idx16_v], add=True)

  plsc.subcore_barrier()

  for k in range(_ROWS_PER_TILE // _ZROWS):
    r = row0 + k * _ZROWS
    pltpu.sync_copy(acc.at[pl.ds(r, _ZROWS)], out_hbm.at[c, pl.ds(r, _ZROWS)])


def _combine_body(p_ref, o_ref):
  o_ref[...] = p_ref[0] + p_ref[1]


def kernel(node_features, gathered_nodes, edge_list):
  del node_features
  dst = edge_list[:, 1]

  mesh = plsc.VectorSubcoreMesh(core_axis_name="c", subcore_axis_name="s")
  sc = pl.kernel(
      _sc_scatter_add,
      out_type=jax.ShapeDtypeStruct((_NC, _N_PAD, _D), jnp.float32),
      mesh=mesh,
      scratch_types=[
          pltpu.VMEM_SHARED((_N_PAD, _D), jnp.float32),
          pltpu.VMEM((_CHUNK, _D), jnp.float32),
          pltpu.VMEM((_CHUNK,), jnp.int32),
          pltpu.VMEM((_REM, _D), jnp.float32),
          pltpu.VMEM((_REM,), jnp.int32),
          pltpu.VMEM((_ZROWS, _D), jnp.float32),
      ],
  )
  partials = sc(gathered_nodes, dst)

  rows_blk = 1000
  out = pl.pallas_call(
      _combine_body,
      out_shape=jax.ShapeDtypeStruct((_N_NODES, _D), jnp.float32),
      grid=(_N_NODES // rows_blk,),
      in_specs=[pl.BlockSpec((_NC, rows_blk, _D), lambda i: (0, i, 0))],
      out_specs=pl.BlockSpec((rows_blk, _D), lambda i: (i, 0)),
  )(partials)
  return out


# speedup vs baseline: 8.8369x; 1.7386x over previous
import functools

import jax
import jax.numpy as jnp
from jax import lax
from jax.experimental import pallas as pl
from jax.experimental.pallas import tpu as pltpu
from jax.experimental.pallas import tpu_sc as plsc

_N_NODES = 10000
_N_EDGES = 320000
_D = 128

_NC = 2
_NS = 16
_NW = _NC * _NS

_EPT = _N_EDGES // _NW
_CHUNK = 128
_FULL_CHUNKS = _EPT // _CHUNK
_REM = _EPT - _FULL_CHUNKS * _CHUNK

_N_PAD = 10240
_ROWS_PER_TILE = _N_PAD // _NS
_ZROWS = 128


_PAIRS = _FULL_CHUNKS // 2


def _sc_scatter_add(gathered_hbm, dst_hbm, out_hbm, acc, feat_a, idx_a,
                    feat_b, idx_b, feat16_v, idx16_v,
                    sem_fa, sem_ia, sem_fb, sem_ib):
  c = lax.axis_index("c")
  s = lax.axis_index("s")
  w = c * _NS + s

  z = jnp.zeros((16,), jnp.float32)

  def zero_row(r, carry):
    for l in range(_D // 16):
      feat_a[r, pl.ds(16 * l, 16)] = z
    return carry

  lax.fori_loop(0, _ZROWS, zero_row, 0)
  row0 = s * _ROWS_PER_TILE
  for k in range(_ROWS_PER_TILE // _ZROWS):
    pltpu.sync_copy(feat_a, acc.at[pl.ds(row0 + k * _ZROWS, _ZROWS)])
  plsc.subcore_barrier()

  base = w * _EPT

  def _start(e0, feat, idx, sem_f, sem_i):
    pltpu.async_copy(dst_hbm.at[pl.ds(e0, _CHUNK)], idx, sem_i)
    pltpu.async_copy(gathered_hbm.at[pl.ds(e0, _CHUNK), 0], feat, sem_f)

  def _wait(e0, feat, idx, sem_f, sem_i):
    pltpu.make_async_copy(dst_hbm.at[pl.ds(e0, _CHUNK)], idx, sem_i).wait()
    pltpu.make_async_copy(gathered_hbm.at[pl.ds(e0, _CHUNK), 0], feat,
                          sem_f).wait()

  _start(pl.multiple_of(base, 16), feat_a, idx_a, sem_fa, sem_ia)

  def chunk_pair(j, carry):
    e_a = pl.multiple_of(base + (2 * j) * _CHUNK, 16)
    e_b = pl.multiple_of(base + (2 * j + 1) * _CHUNK, 16)
    _start(e_b, feat_b, idx_b, sem_fb, sem_ib)
    _wait(e_a, feat_a, idx_a, sem_fa, sem_ia)
    pltpu.sync_copy(feat_a, acc.at[idx_a], add=True)

    @pl.when(j < _PAIRS - 1)
    def _():
      e_a2 = pl.multiple_of(base + (2 * j + 2) * _CHUNK, 16)
      _start(e_a2, feat_a, idx_a, sem_fa, sem_ia)

    _wait(e_b, feat_b, idx_b, sem_fb, sem_ib)
    pltpu.sync_copy(feat_b, acc.at[idx_b], add=True)
    return carry

  lax.fori_loop(0, _PAIRS, chunk_pair, 0)

  e0 = pl.multiple_of(base + _FULL_CHUNKS * _CHUNK, 8)
  pltpu.sync_copy(dst_hbm.at[pl.ds(e0, _REM)], idx16_v)
  pltpu.sync_copy(gathered_hbm.at[pl.ds(e0, _REM), 0], feat16_v)
  pltpu.sync_copy(feat16_v, acc.at[idx16_v], add=True)

  plsc.subcore_barrier()

  for k in range(_ROWS_PER_TILE // _ZROWS):
    r = row0 + k * _ZROWS
    pltpu.sync_copy(acc.at[pl.ds(r, _ZROWS)], out_hbm.at[c, pl.ds(r, _ZROWS)])


def _combine_body(p_ref, o_ref):
  o_ref[...] = p_ref[0] + p_ref[1]


def kernel(node_features, gathered_nodes, edge_list):
  del node_features
  dst = edge_list[:, 1]

  mesh = plsc.VectorSubcoreMesh(core_axis_name="c", subcore_axis_name="s")
  sc = pl.kernel(
      _sc_scatter_add,
      out_type=jax.ShapeDtypeStruct((_NC, _N_PAD, _D), jnp.float32),
      mesh=mesh,
      scratch_types=[
          pltpu.VMEM_SHARED((_N_PAD, _D), jnp.float32),
          pltpu.VMEM((_CHUNK, _D), jnp.float32),
          pltpu.VMEM((_CHUNK,), jnp.int32),
          pltpu.VMEM((_CHUNK, _D), jnp.float32),
          pltpu.VMEM((_CHUNK,), jnp.int32),
          pltpu.VMEM((_REM, _D), jnp.float32),
          pltpu.VMEM((_REM,), jnp.int32),
          pltpu.SemaphoreType.DMA,
          pltpu.SemaphoreType.DMA,
          pltpu.SemaphoreType.DMA,
          pltpu.SemaphoreType.DMA,
      ],
  )
  partials = sc(gathered_nodes, dst)

  rows_blk = 1000
  out = pl.pallas_call(
      _combine_body,
      out_shape=jax.ShapeDtypeStruct((_N_NODES, _D), jnp.float32),
      grid=(_N_NODES // rows_blk,),
      in_specs=[pl.BlockSpec((_NC, rows_blk, _D), lambda i: (0, i, 0))],
      out_specs=pl.BlockSpec((rows_blk, _D), lambda i: (i, 0)),
  )(partials)
  return out
